# Initial kernel scaffold; baseline (speedup 1.0000x reference)
import jax
import jax.numpy as jnp
from jax.experimental import pallas as pl


def kernel(inc_0_w, inc_0_scale, inc_0_bias, inc_1_w, inc_1_scale, inc_1_bias, down1_0_w, down1_0_scale, down1_0_bias, down1_1_w, down1_1_scale, down1_1_bias, down2_0_w, down2_0_scale, down2_0_bias, down2_1_w, down2_1_scale, down2_1_bias, down3_0_w, down3_0_scale, down3_0_bias, down3_1_w, down3_1_scale, down3_1_bias, down4_0_w, down4_0_scale, down4_0_bias, down4_1_w, down4_1_scale, down4_1_bias, up1_0_w, up1_0_scale, up1_0_bias, up1_1_w, up1_1_scale, up1_1_bias, up2_0_w, up2_0_scale, up2_0_bias, up2_1_w, up2_1_scale, up2_1_bias, up3_0_w, up3_0_scale, up3_0_bias, up3_1_w, up3_1_scale, up3_1_bias, up4_0_w, up4_0_scale, up4_0_bias, up4_1_w, up4_1_scale, up4_1_bias, outc_w, outc_b, a_old, p_old, mask_flow, v_cond, mask_cond):
    raise NotImplementedError("write your pallas kernel here")


# speedup vs baseline: 3.0620x; 3.0620x over previous
import jax
import jax.numpy as jnp
from jax.experimental import pallas as pl
from jax.experimental.pallas import tpu as pltpu

_BF = jnp.bfloat16
_F32 = jnp.float32
_TARGET_M = 2048


def _double_conv_stage(inputs, wm1, b1, wm2, b2, preshifted=False):
    N = inputs[0].shape[0]
    if preshifted:
        H, W = inputs[0].shape[1] - 2, inputs[0].shape[2]
        Cs = [inputs[0].shape[3] // 3]
    else:
        H, W = inputs[0].shape[1], inputs[0].shape[2]
        Cs = [x.shape[-1] for x in inputs]
    Cmid = wm1.shape[1]
    Cout = wm2.shape[1]
    TH = min(H, max(1, _TARGET_M // W))
    n_in = len(inputs)

    def body(*refs):
        x_refs = refs[:n_in]
        wm1_ref, b1_ref, wm2_ref, b2_ref, o_ref = refs[n_in:n_in + 5]
        scratch = refs[n_in + 5:]
        if preshifted:
            xs_refs = []
            xsm_ref = scratch[0]
        else:
            xs_refs = scratch[:n_in]
            xsm_ref = scratch[n_in]

        for x_ref, xs_ref in zip(x_refs, xs_refs):
            x = x_ref[0]
            C = x.shape[-1]
            zrow = jnp.zeros((3, 1, W, C), _BF)
            zcol = jnp.zeros((H, 1, C), _BF)
            xs_ref[:, 0:1] = zrow
            xs_ref[:, H + 1:H + 2] = zrow
            xs_ref[1, 1:H + 1] = x
            xs_ref[0, 1:H + 1, 0:1] = zcol
            xs_ref[0, 1:H + 1, 1:W] = x[:, :W - 1]
            xs_ref[2, 1:H + 1, W - 1:W] = zcol
            xs_ref[2, 1:H + 1, 0:W - 1] = x[:, 1:]

        zrow_m = jnp.zeros((3, 1, W, Cmid), _BF)
        zcol_m = jnp.zeros((H, 1, Cmid), _BF)
        xsm_ref[:, 0:1] = zrow_m
        xsm_ref[:, H + 1:H + 2] = zrow_m
        xsm_ref[0, 1:H + 1, 0:1] = zcol_m
        xsm_ref[2, 1:H + 1, W - 1:W] = zcol_m

        w1v = wm1_ref[...]
        b1v = b1_ref[...]
        for r0 in range(0, H, TH):
            pieces = []
            if preshifted:
                C3 = 3 * Cs[0]
                for dy in range(3):
                    pieces.append(
                        x_refs[0][0, r0 + dy:r0 + dy + TH].reshape(TH * W, C3))
            else:
                for xs_ref, C in zip(xs_refs, Cs):
                    for dy in range(3):
                        for dx in range(3):
                            pieces.append(
                                xs_ref[dx, r0 + dy:r0 + dy + TH]
                                .reshape(TH * W, C))
            xcat = jnp.concatenate(pieces, axis=-1) if len(pieces) > 1 else pieces[0]
            acc = jnp.dot(xcat, w1v, preferred_element_type=_F32)
            y = jnp.maximum(acc + b1v, 0.0).astype(_BF).reshape(TH, W, Cmid)
            xsm_ref[1, 1 + r0:1 + r0 + TH] = y
            xsm_ref[0, 1 + r0:1 + r0 + TH, 1:W] = y[:, :W - 1]
            xsm_ref[2, 1 + r0:1 + r0 + TH, 0:W - 1] = y[:, 1:]

        w2v = wm2_ref[...]
        b2v = b2_ref[...]
        for r0 in range(0, H, TH):
            pieces = [
                xsm_ref[dx, r0 + dy:r0 + dy + TH].reshape(TH * W, Cmid)
                for dy in range(3) for dx in range(3)
            ]
            xcat = jnp.concatenate(pieces, axis=-1)
            acc = jnp.dot(xcat, w2v, preferred_element_type=_F32)
            z = jnp.maximum(acc + b2v, 0.0).astype(_BF)
            o_ref[0, r0:r0 + TH] = z.reshape(TH, W, Cout)

    in_specs = []
    for x in inputs:
        shp = x.shape
        in_specs.append(
            pl.BlockSpec((1,) + shp[1:], lambda n: (n, 0, 0, 0)))
    for wv in (wm1, b1, wm2, b2):
        in_specs.append(pl.BlockSpec(wv.shape, lambda n: (0, 0)))

    scratch_shapes = []
    if not preshifted:
        for C in Cs:
            scratch_shapes.append(pltpu.VMEM((3, H + 2, W, C), _BF))
    scratch_shapes.append(pltpu.VMEM((3, H + 2, W, Cmid), _BF))

    return pl.pallas_call(
        body,
        out_shape=jax.ShapeDtypeStruct((N, H, W, Cout), _BF),
        grid=(N,),
        in_specs=in_specs,
        out_specs=pl.BlockSpec((1, H, W, Cout), lambda n: (n, 0, 0, 0)),
        scratch_shapes=scratch_shapes,
        compiler_params=pltpu.CompilerParams(
            dimension_semantics=("parallel",),
            vmem_limit_bytes=100 * 1024 * 1024),
    )(*inputs, wm1, b1, wm2, b2)


def _pack_w(w, scale):
    return (w * scale).reshape(-1, w.shape[-1]).astype(_BF)


def _pack_w_split(w, scale, c_skip):
    wa = (w[:, :, :c_skip] * scale).reshape(-1, w.shape[-1])
    wb = (w[:, :, c_skip:] * scale).reshape(-1, w.shape[-1])
    return jnp.concatenate([wa, wb], axis=0).astype(_BF)


def _brow(b):
    return b.reshape(1, -1).astype(_F32)


def _pool(x):
    N, H, W, C = x.shape
    return x.reshape(N, H // 2, 2, W // 2, 2, C).max(axis=(2, 4))


def _interp_mat(n):
    if n == 1:
        return jnp.ones((2, 1), _F32)
    src = jnp.arange(2 * n, dtype=_F32) * (n - 1) / (2 * n - 1)
    i0 = jnp.clip(jnp.floor(src).astype(jnp.int32), 0, n - 1)
    i1 = jnp.minimum(i0 + 1, n - 1)
    f = src - i0.astype(_F32)
    return (jax.nn.one_hot(i0, n, dtype=_F32) * (1.0 - f)[:, None]
            + jax.nn.one_hot(i1, n, dtype=_F32) * f[:, None])


def _ups(x):
    N, H, W, C = x.shape
    y = jnp.einsum("oh,nhwc->nowc", _interp_mat(H), x.astype(_F32))
    y = jnp.einsum("pw,nowc->nopc", _interp_mat(W), y)
    return y.astype(x.dtype)


def kernel(
    inc_0_w, inc_0_scale, inc_0_bias, inc_1_w, inc_1_scale, inc_1_bias,
    down1_0_w, down1_0_scale, down1_0_bias, down1_1_w, down1_1_scale, down1_1_bias,
    down2_0_w, down2_0_scale, down2_0_bias, down2_1_w, down2_1_scale, down2_1_bias,
    down3_0_w, down3_0_scale, down3_0_bias, down3_1_w, down3_1_scale, down3_1_bias,
    down4_0_w, down4_0_scale, down4_0_bias, down4_1_w, down4_1_scale, down4_1_bias,
    up1_0_w, up1_0_scale, up1_0_bias, up1_1_w, up1_1_scale, up1_1_bias,
    up2_0_w, up2_0_scale, up2_0_bias, up2_1_w, up2_1_scale, up2_1_bias,
    up3_0_w, up3_0_scale, up3_0_bias, up3_1_w, up3_1_scale, up3_1_bias,
    up4_0_w, up4_0_scale, up4_0_bias, up4_1_w, up4_1_scale, up4_1_bias,
    outc_w, outc_b,
    a_old, p_old, mask_flow, v_cond, mask_cond):
    to_nhwc = lambda t: jnp.transpose(t, (0, 2, 3, 1))
    a = to_nhwc(a_old)
    p = to_nhwc(p_old)
    mf = to_nhwc(mask_flow)
    vc = to_nhwc(v_cond)
    mc = to_nhwc(mask_cond)

    vdy = jnp.pad(a[:, 1:], ((0, 0), (0, 1), (0, 0), (0, 0))) - a
    vdx = jnp.pad(a[:, :, 1:], ((0, 0), (0, 0), (0, 1), (0, 0))) - a
    v_old = jnp.concatenate([vdy, -vdx], axis=-1)
    x13 = jnp.concatenate(
        [p, a, v_old, mf, vc * mc, mc, mf * p, mf * v_old, v_old * mc],
        axis=-1).astype(_BF)
    N, H, W, _ = x13.shape

    xp = jnp.pad(x13, ((0, 0), (1, 1), (1, 1), (0, 0)))
    x39 = jnp.concatenate([xp[:, :, d:d + W] for d in range(3)], axis=-1)

    x1 = _double_conv_stage(
        [x39], _pack_w(inc_0_w, inc_0_scale), _brow(inc_0_bias),
        _pack_w(inc_1_w, inc_1_scale), _brow(inc_1_bias), preshifted=True)
    x2 = _double_conv_stage(
        [_pool(x1)], _pack_w(down1_0_w, down1_0_scale), _brow(down1_0_bias),
        _pack_w(down1_1_w, down1_1_scale), _brow(down1_1_bias))
    x3 = _double_conv_stage(
        [_pool(x2)], _pack_w(down2_0_w, down2_0_scale), _brow(down2_0_bias),
        _pack_w(down2_1_w, down2_1_scale), _brow(down2_1_bias))
    x4 = _double_conv_stage(
        [_pool(x3)], _pack_w(down3_0_w, down3_0_scale), _brow(down3_0_bias),
        _pack_w(down3_1_w, down3_1_scale), _brow(down3_1_bias))
    x5 = _double_conv_stage(
        [_pool(x4)], _pack_w(down4_0_w, down4_0_scale), _brow(down4_0_bias),
        _pack_w(down4_1_w, down4_1_scale), _brow(down4_1_bias))

    y = _double_conv_stage(
        [x4, _ups(x5)],
        _pack_w_split(up1_0_w, up1_0_scale, x4.shape[-1]), _brow(up1_0_bias),
        _pack_w(up1_1_w, up1_1_scale), _brow(up1_1_bias))
    y = _double_conv_stage(
        [x3, _ups(y)],
        _pack_w_split(up2_0_w, up2_0_scale, x3.shape[-1]), _brow(up2_0_bias),
        _pack_w(up2_1_w, up2_1_scale), _brow(up2_1_bias))
    y = _double_conv_stage(
        [x2, _ups(y)],
        _pack_w_split(up3_0_w, up3_0_scale, x2.shape[-1]), _brow(up3_0_bias),
        _pack_w(up3_1_w, up3_1_scale), _brow(up3_1_bias))
    y = _double_conv_stage(
        [jnp.concatenate([x1, _ups(y)], axis=-1)],
        _pack_w(up4_0_w, up4_0_scale), _brow(up4_0_bias),
        _pack_w(up4_1_w, up4_1_scale), _brow(up4_1_bias))

    o = jnp.einsum("nhwc,cd->nhwd", y.astype(_F32), outc_w,
                   preferred_element_type=_F32) + outc_b[None, None, None, :]
    a_new = 400.0 * jnp.tanh((a + o[..., 0:1]) / 400.0)
    p_new = 10.0 * jnp.tanh((p + o[..., 1:2]) / 10.0)
    from_nhwc = lambda t: jnp.transpose(t, (0, 3, 1, 2))
    return from_nhwc(a_new), from_nhwc(p_new)
